# baseline (speedup 1.0000x reference)
import functools

import jax
import jax.numpy as jnp
from jax import lax
from jax.experimental import pallas as pl
from jax.experimental.pallas import tpu as pltpu
from jax.experimental.pallas import tpu_sc as plsc

_B = 16
_N = 20000
_C = 80
_M = 1000
_S = 8000
_T = _B * _M
_CH = 128
_NCHUNK = 8
_LEN = _CH * _NCHUNK


def _gather16(x, idx):
    dnums = lax.GatherDimensionNumbers(
        offset_dims=(), collapsed_slice_dims=(0,), start_index_map=(0,))
    return lax.gather(x, idx[:, None], dnums, (1,),
                      mode=lax.GatherScatterMode.PROMISE_IN_BOUNDS)


def _cumsum16(x):
    idx = lax.iota(jnp.int32, 16)
    y = x
    for sh in (1, 2, 4, 8):
        src = jnp.where(idx >= sh, idx - sh, 0)
        g = _gather16(y, src)
        y = y + jnp.where(idx >= sh, g, 0)
    return y


def _lane15(x):
    return _gather16(x, jnp.full((16,), 15, jnp.int32))


def _sc_body(sel1d, boxes1d, scores1d,
             counts, oboxes, oscores, oclasses,
             sel_v, list_v, bidx_v, sidx_v,
             bcol, sbuf, cbuf, ibuf, cnt_ref, gsem):
    cid = lax.axis_index("c")
    sid = lax.axis_index("s")
    wid = sid * 2 + cid

    @pl.when(wid < _B)
    def _():
        b = wid
        pltpu.sync_copy(sel1d, sel_v)

        zero16 = jnp.zeros((16,), jnp.int32)
        for g in range(_LEN // 16):
            list_v[pl.ds(g * 16, 16)] = zero16
        cnt_ref[...] = zero16

        base_iota = lax.iota(jnp.int32, 16)

        @pl.loop(0, _S // 16)
        def _filter(g):
            bv = sel_v[pl.ds(g * 16, 16)]
            m = bv == b
            mi = jnp.where(m, jnp.int32(1), jnp.int32(0))
            cv = cnt_ref[...]
            cs = _cumsum16(mi)
            pos = cv + cs - 1
            pos = jnp.where(m, pos, _S + base_iota)
            plsc.store_scatter(list_v, [pos], base_iota + g * 16)
            cnt_ref[...] = cv + _lane15(cs)

        cnt_vec = cnt_ref[...]
        cap_vec = jnp.minimum(cnt_vec, _M)

        for g in range(_LEN // 16):
            r0 = g * 16
            rvec = base_iota + r0
            valid = rvec < cap_vec
            rid = list_v[pl.ds(r0, 16)]
            x = plsc.load_gather(sel_v, [rid + 2 * _S])
            l = plsc.load_gather(sel_v, [rid + _S])
            bi4 = (b * _N + x) * 4
            j, k = divmod(g, _CH // 16)
            sidx_v[j, pl.ds(k * 16, 16)] = (b * _N + x) * _C + l
            for c in range(4):
                bidx_v[4 * j + c, pl.ds(k * 16, 16)] = bi4 + c
            cbuf[pl.ds(r0, 16)] = jnp.where(valid, l, 0)

        copies = []
        for j in range(_NCHUNK):
            copies.append(pltpu.async_copy(
                scores1d.at[sidx_v.at[j]], sbuf.at[pl.ds(j * _CH, _CH)], gsem))
            for c in range(4):
                copies.append(pltpu.async_copy(
                    boxes1d.at[bidx_v.at[4 * j + c]],
                    bcol.at[pl.ds(c * _LEN + j * _CH, _CH)], gsem))
        for cp in copies:
            cp.wait()

        zf16 = jnp.zeros((16,), jnp.float32)
        for g in range(_LEN // 16):
            r0 = g * 16
            rvec = base_iota + r0
            invalid = rvec >= cap_vec
            sl = pl.ds(r0, 16)
            sbuf[sl] = jnp.where(invalid, zf16, sbuf[sl])
            for c in range(4):
                v = bcol[pl.ds(c * _LEN + r0, 16)]
                v = jnp.where(invalid, zf16, v)
                plsc.store_scatter(ibuf, [rvec * 4 + c], v)

        pltpu.sync_copy(ibuf.at[pl.ds(0, 4 * _M)],
                        oboxes.at[pl.ds(b * 4 * _M, 4 * _M)])
        pltpu.sync_copy(sbuf.at[pl.ds(0, _M)], oscores.at[pl.ds(b * _M, _M)])
        pltpu.sync_copy(cbuf.at[pl.ds(0, _M)], oclasses.at[pl.ds(b * _M, _M)])
        pltpu.sync_copy(cnt_ref, counts.at[b])


_sc_kernel = functools.partial(
    pl.kernel,
    out_type=[
        jax.ShapeDtypeStruct((_B, 16), jnp.int32),
        jax.ShapeDtypeStruct((_T * 4,), jnp.float32),
        jax.ShapeDtypeStruct((_T,), jnp.float32),
        jax.ShapeDtypeStruct((_T,), jnp.int32),
    ],
    mesh=plsc.VectorSubcoreMesh(core_axis_name="c", subcore_axis_name="s"),
    compiler_params=pltpu.CompilerParams(needs_layout_passes=False),
    scratch_types=[
        pltpu.VMEM((3 * _S,), jnp.int32),
        pltpu.VMEM((_S + 16,), jnp.int32),
        pltpu.VMEM((4 * _NCHUNK, _CH), jnp.int32),
        pltpu.VMEM((_NCHUNK, _CH), jnp.int32),
        pltpu.VMEM((4 * _LEN,), jnp.float32),
        pltpu.VMEM((_LEN,), jnp.float32),
        pltpu.VMEM((_LEN,), jnp.int32),
        pltpu.VMEM((4 * _LEN,), jnp.float32),
        pltpu.VMEM((16,), jnp.int32),
        pltpu.SemaphoreType.DMA,
    ],
)(_sc_body)


def kernel(pred_boxes, pred_scores, selected_indexes):
    sel1d = selected_indexes.astype(jnp.int32).T.reshape(3 * _S)
    boxes1d = pred_boxes.reshape(_B * _N * 4)
    scores1d = pred_scores.reshape(_B * _N * _C)
    counts, oboxes, oscores, oclasses = _sc_kernel(sel1d, boxes1d, scores1d)
    return (counts[:, :1],
            oboxes.reshape(_B, _M, 4),
            oscores.reshape(_B, _M),
            oclasses.reshape(_B, _M))


# speedup vs baseline: 2.8981x; 2.8981x over previous
import functools

import jax
import jax.numpy as jnp
from jax import lax
from jax.experimental import pallas as pl
from jax.experimental.pallas import tpu as pltpu
from jax.experimental.pallas import tpu_sc as plsc

_B = 16
_N = 20000
_C = 80
_M = 1000
_S = 8000
_T = _B * _M
_H = 512
_NG = _H // 16
_NCH = _H // 16
_LDUMP = 1024


def _gather16(x, idx):
    dnums = lax.GatherDimensionNumbers(
        offset_dims=(), collapsed_slice_dims=(0,), start_index_map=(0,))
    return lax.gather(x, idx[:, None], dnums, (1,),
                      mode=lax.GatherScatterMode.PROMISE_IN_BOUNDS)


def _cumsum16(x):
    idx = lax.iota(jnp.int32, 16)
    y = x
    for sh in (1, 2, 4, 8):
        src = jnp.where(idx >= sh, idx - sh, 0)
        g = _gather16(y, src)
        y = y + jnp.where(idx >= sh, g, 0)
    return y


def _lane15(x):
    return _gather16(x, jnp.full((16,), 15, jnp.int32))


def _sc_body(sel1d, boxes_t, scores_t,
             counts, oboxes, oscores, oclasses,
             sel_b, list_v, xidx_v, lidx_v, xg, lg,
             pbuf, cbuf, sobuf, sstA, sstB, bstA, bstB, ibuf, cnt_ref,
             gsem, semA, semB):
    b = lax.axis_index("s")
    h = lax.axis_index("c")

    pltpu.sync_copy(sel1d.at[pl.ds(0, _S)], sel_b.at[pl.ds(0, _S)])

    zero16 = jnp.zeros((16,), jnp.int32)
    for g in range(_LDUMP // 16):
        list_v[pl.ds(g * 16, 16)] = zero16
    cnt_ref[...] = zero16

    base_iota = lax.iota(jnp.int32, 16)

    @pl.loop(0, _S // 16)
    def _filter(g):
        bv = sel_b[pl.ds(g * 16, 16)]
        m = bv == b
        mi = jnp.where(m, jnp.int32(1), jnp.int32(0))
        cv = cnt_ref[...]
        cs = _cumsum16(mi)
        pos = cv + cs - 1
        pos = jnp.where(m, pos, _S + base_iota)
        pos = jnp.minimum(pos, _LDUMP + base_iota)
        plsc.store_scatter(list_v, [pos], base_iota + g * 16)
        cnt_ref[...] = cv + _lane15(cs)

    cnt_vec = cnt_ref[...]
    cap_vec = jnp.minimum(cnt_vec, _M)
    r0_glob = h * _H

    for g in range(_NG):
        rid = list_v[pl.ds(r0_glob + g * 16, 16)]
        r, o = divmod(g, 8)
        xidx_v[r, pl.ds(o * 16, 16)] = rid + 2 * _S
        lidx_v[r, pl.ds(o * 16, 16)] = rid + _S
    idx_copies = []
    for r in range(4):
        idx_copies.append(pltpu.async_copy(
            sel1d.at[xidx_v.at[r]], xg.at[pl.ds(r * 128, 128)], gsem))
        idx_copies.append(pltpu.async_copy(
            sel1d.at[lidx_v.at[r]], lg.at[pl.ds(r * 128, 128)], gsem))
    for cp in idx_copies:
        cp.wait()

    for g in range(_NG):
        sl = pl.ds(g * 16, 16)
        x = xg[sl]
        l = lg[sl]
        rvec = r0_glob + g * 16 + base_iota
        valid = rvec < cap_vec
        pbuf[sl] = l * 32768 + x
        cbuf[sl] = jnp.where(valid, l, 0)

    def issue(cc, sst, bst, sem):
        pv = pbuf[pl.ds(cc * 16, 16)]
        lv8 = (pv >> 15) & 120
        xv128 = ((pv & 32767) >> 7) * 128
        for j in range(16):
            l8 = pl.multiple_of(lv8[j], 8)
            x128 = pl.multiple_of(xv128[j], 128)
            pltpu.async_copy(scores_t.at[b, pl.ds(l8, 8), pl.ds(x128, 128)],
                             sst.at[:, pl.ds(j * 128, 128)], sem)
            pltpu.async_copy(boxes_t.at[b, :, pl.ds(x128, 128)],
                             bst.at[:, pl.ds(j * 128, 128)], sem)

    def drain(sst, bst, sem):
        pltpu.make_async_copy(
            scores_t.at[0, pl.ds(0, 8), pl.ds(0, 2048)], sst, sem).wait()
        pltpu.make_async_copy(
            boxes_t.at[0, :, pl.ds(0, 2048)], bst, sem).wait()

    zf16 = jnp.zeros((16,), jnp.float32)

    col_ids = [jnp.full((16,), c, jnp.int32) for c in range(4)]

    def extract(cc, sst, bst):
        sl = pl.ds(cc * 16, 16)
        p = pbuf[sl]
        x = p & 32767
        la = (p >> 15) & 7
        rloc = cc * 16 + base_iota
        invalid = (r0_glob + rloc) >= cap_vec
        spos = base_iota * 128 + (x & 127)
        sval = plsc.load_gather(sst, [la, spos])
        plsc.store_scatter(sobuf, [rloc], jnp.where(invalid, zf16, sval))
        for c in range(4):
            bval = plsc.load_gather(bst, [col_ids[c], spos])
            bval = jnp.where(invalid, zf16, bval)
            plsc.store_scatter(ibuf, [rloc * 4 + c], bval)

    bufs = [(sstA, bstA, semA), (sstB, bstB, semB)]
    issue(0, *bufs[0])
    for cc in range(_NCH):
        if cc + 1 < _NCH:
            issue(cc + 1, *bufs[(cc + 1) % 2])
        drain(*bufs[cc % 2])
        extract(cc, bufs[cc % 2][0], bufs[cc % 2][1])

    @pl.when(h == 0)
    def _():
        pltpu.sync_copy(ibuf.at[pl.ds(0, 4 * _H)],
                        oboxes.at[pl.ds(b * 4 * _M, 4 * _H)])
        pltpu.sync_copy(sobuf.at[pl.ds(0, _H)],
                        oscores.at[pl.ds(b * _M, _H)])
        pltpu.sync_copy(cbuf.at[pl.ds(0, _H)],
                        oclasses.at[pl.ds(b * _M, _H)])
        pltpu.sync_copy(cnt_ref, counts.at[b])

    @pl.when(h == 1)
    def _():
        rem = _M - _H
        pltpu.sync_copy(ibuf.at[pl.ds(0, 4 * rem)],
                        oboxes.at[pl.ds(b * 4 * _M + 4 * _H, 4 * rem)])
        pltpu.sync_copy(sobuf.at[pl.ds(0, rem)],
                        oscores.at[pl.ds(b * _M + _H, rem)])
        pltpu.sync_copy(cbuf.at[pl.ds(0, rem)],
                        oclasses.at[pl.ds(b * _M + _H, rem)])


_sc_kernel = functools.partial(
    pl.kernel,
    out_type=[
        jax.ShapeDtypeStruct((_B, 16), jnp.int32),
        jax.ShapeDtypeStruct((_T * 4,), jnp.float32),
        jax.ShapeDtypeStruct((_T,), jnp.float32),
        jax.ShapeDtypeStruct((_T,), jnp.int32),
    ],
    mesh=plsc.VectorSubcoreMesh(core_axis_name="c", subcore_axis_name="s"),
    compiler_params=pltpu.CompilerParams(needs_layout_passes=False),
    scratch_types=[
        pltpu.VMEM((_S + 16,), jnp.int32),
        pltpu.VMEM((_LDUMP + 32,), jnp.int32),
        pltpu.VMEM((4, 128), jnp.int32),
        pltpu.VMEM((4, 128), jnp.int32),
        pltpu.VMEM((_H,), jnp.int32),
        pltpu.VMEM((_H,), jnp.int32),
        pltpu.VMEM((_H,), jnp.int32),
        pltpu.VMEM((_H,), jnp.int32),
        pltpu.VMEM((_H,), jnp.float32),
        pltpu.VMEM((8, 2048), jnp.float32),
        pltpu.VMEM((8, 2048), jnp.float32),
        pltpu.VMEM((4, 2048), jnp.float32),
        pltpu.VMEM((4, 2048), jnp.float32),
        pltpu.VMEM((_H * 4,), jnp.float32),
        pltpu.VMEM((16,), jnp.int32),
        pltpu.SemaphoreType.DMA,
        pltpu.SemaphoreType.DMA,
        pltpu.SemaphoreType.DMA,
    ],
)(_sc_body)


def kernel(pred_boxes, pred_scores, selected_indexes):
    sel1d = selected_indexes.astype(jnp.int32).T.reshape(3 * _S)
    boxes_t = jnp.transpose(pred_boxes, (0, 2, 1))
    scores_t = jnp.transpose(pred_scores, (0, 2, 1))
    counts, oboxes, oscores, oclasses = _sc_kernel(sel1d, boxes_t, scores_t)
    return (counts[:, :1],
            oboxes.reshape(_B, _M, 4),
            oscores.reshape(_B, _M),
            oclasses.reshape(_B, _M))


# speedup vs baseline: 3.7699x; 1.3008x over previous
import functools

import jax
import jax.numpy as jnp
from jax import lax
from jax.experimental import pallas as pl
from jax.experimental.pallas import tpu as pltpu
from jax.experimental.pallas import tpu_sc as plsc

_B = 16
_N = 20000
_C = 80
_M = 1000
_S = 8000
_T = _B * _M
_R = 1024
_NG = _R // 16
_NCH = 32
_LDUMP = 1024


def _gather16(x, idx):
    dnums = lax.GatherDimensionNumbers(
        offset_dims=(), collapsed_slice_dims=(0,), start_index_map=(0,))
    return lax.gather(x, idx[:, None], dnums, (1,),
                      mode=lax.GatherScatterMode.PROMISE_IN_BOUNDS)


def _cumsum16(x):
    idx = lax.iota(jnp.int32, 16)
    y = x
    for sh in (1, 2, 4, 8):
        src = jnp.where(idx >= sh, idx - sh, 0)
        g = _gather16(y, src)
        y = y + jnp.where(idx >= sh, g, 0)
    return y


def _lane15(x):
    return _gather16(x, jnp.full((16,), 15, jnp.int32))


def _sc_body(sel1d, boxes_t, scores_t,
             counts, oboxes, oscores, oclasses,
             sel_b, list_v, xidx_v, lidx_v, xg, lg,
             pbuf, cbuf, sobuf, sstA, sstB, bstA, bstB, ibuf, cnt_ref,
             gsem, semA, semB):
    b = lax.axis_index("s")
    h = lax.axis_index("c")
    zero16 = jnp.zeros((16,), jnp.int32)
    zf16 = jnp.zeros((16,), jnp.float32)

    @pl.loop(0, _R // 16)
    def _z1(g):
        sobuf[pl.ds(g * 16, 16)] = zf16
        cbuf[pl.ds(g * 16, 16)] = zero16

    @pl.loop(0, 4 * _R // 16)
    def _z2(g):
        ibuf[pl.ds(g * 16, 16)] = zf16

    pltpu.sync_copy(sel1d.at[pl.ds(0, _S)], sel_b.at[pl.ds(0, _S)])

    @pl.loop(0, _LDUMP // 16)
    def _z3(g):
        list_v[pl.ds(g * 16, 16)] = zero16
    cnt_ref[...] = zero16

    base_iota = lax.iota(jnp.int32, 16)

    @pl.loop(0, _S // 16)
    def _filter(g):
        bv = sel_b[pl.ds(g * 16, 16)]
        m = bv == b
        mi = jnp.where(m, jnp.int32(1), jnp.int32(0))
        cv = cnt_ref[...]
        cs = _cumsum16(mi)
        pos = cv + cs - 1
        pos = jnp.where(m, pos, _S + base_iota)
        pos = jnp.minimum(pos, _LDUMP + base_iota)
        plsc.store_scatter(list_v, [pos], base_iota + g * 16)
        cnt_ref[...] = cv + _lane15(cs)

    cnt_vec = cnt_ref[...]
    cap_vec = jnp.minimum(cnt_vec, _M)
    cap_s = cap_vec[0]

    @pl.loop(0, _NG)
    def _bidx(g):
        rid = list_v[pl.ds(g * 16, 16)]
        xidx_v[g >> 3, pl.ds((g & 7) * 16, 16)] = rid + 2 * _S
        lidx_v[g >> 3, pl.ds((g & 7) * 16, 16)] = rid + _S
    idx_copies = []
    for r in range(8):
        idx_copies.append(pltpu.async_copy(
            sel1d.at[xidx_v.at[r]], xg.at[pl.ds(r * 128, 128)], gsem))
        idx_copies.append(pltpu.async_copy(
            sel1d.at[lidx_v.at[r]], lg.at[pl.ds(r * 128, 128)], gsem))
    for cp in idx_copies:
        cp.wait()

    @pl.loop(0, _NG)
    def _pack(g):
        sl = pl.ds(g * 16, 16)
        x = xg[sl]
        l = lg[sl]
        rvec = g * 16 + base_iota
        valid = rvec < cap_vec
        pbuf[sl] = jnp.where(valid, l * 32768 + x, (rvec * 128) & 16383)
        cbuf[sl] = jnp.where(valid, l, 0)

    def issue(k, sst, bst, sem):
        gr = (2 * k + h) * 16
        pv = pbuf[pl.ds(gr, 16)]
        lv8 = (pv >> 15) & 120
        xv128 = ((pv & 32767) >> 7) * 128
        for j in range(16):
            l8 = pl.multiple_of(lv8[j], 8)
            x128 = pl.multiple_of(xv128[j], 128)
            pltpu.async_copy(scores_t.at[b, pl.ds(l8, 8), pl.ds(x128, 128)],
                             sst.at[:, pl.ds(j * 128, 128)], sem)
            pltpu.async_copy(boxes_t.at[b, :, pl.ds(x128, 128)],
                             bst.at[:, pl.ds(j * 128, 128)], sem)

    def drain(sst, bst, sem):
        pltpu.make_async_copy(
            scores_t.at[0, pl.ds(0, 8), pl.ds(0, 2048)], sst, sem).wait()
        pltpu.make_async_copy(
            boxes_t.at[0, :, pl.ds(0, 2048)], bst, sem).wait()

    col_ids = [jnp.full((16,), c, jnp.int32) for c in range(4)]

    def extract(k, sst, bst):
        gr = (2 * k + h) * 16
        p = pbuf[pl.ds(gr, 16)]
        x = p & 32767
        la = (p >> 15) & 7
        rvec = gr + base_iota
        invalid = rvec >= cap_vec
        spos = base_iota * 128 + (x & 127)
        sval = plsc.load_gather(sst, [la, spos])
        plsc.store_scatter(sobuf, [rvec], jnp.where(invalid, zf16, sval))
        for c in range(4):
            bval = plsc.load_gather(bst, [col_ids[c], spos])
            bval = jnp.where(invalid, zf16, bval)
            plsc.store_scatter(ibuf, [rvec * 4 + c], bval)

    def active(k):
        return (2 * k + h) * 16 < cap_s

    bufs = [(sstA, bstA, semA), (sstB, bstB, semB)]

    @pl.when(active(0))
    def _():
        issue(0, *bufs[0])

    for cc in range(_NCH):
        if cc + 1 < _NCH:
            @pl.when(active(cc + 1))
            def _(cc=cc):
                issue(cc + 1, *bufs[(cc + 1) % 2])

        @pl.when(active(cc))
        def _(cc=cc):
            drain(*bufs[cc % 2])
            extract(cc, bufs[cc % 2][0], bufs[cc % 2][1])

    @pl.when(h == 0)
    def _():
        for k in range(32):
            gr = 32 * k
            n = 16 if gr + 16 <= _M else _M - gr
            pltpu.sync_copy(ibuf.at[pl.ds(4 * gr, 4 * n)],
                            oboxes.at[pl.ds(b * 4 * _M + 4 * gr, 4 * n)])
            pltpu.sync_copy(sobuf.at[pl.ds(gr, n)],
                            oscores.at[pl.ds(b * _M + gr, n)])
            pltpu.sync_copy(cbuf.at[pl.ds(gr, n)],
                            oclasses.at[pl.ds(b * _M + gr, n)])
        pltpu.sync_copy(cnt_ref, counts.at[b])

    @pl.when(h == 1)
    def _():
        for k in range(31):
            gr = 32 * k + 16
            pltpu.sync_copy(ibuf.at[pl.ds(4 * gr, 64)],
                            oboxes.at[pl.ds(b * 4 * _M + 4 * gr, 64)])
            pltpu.sync_copy(sobuf.at[pl.ds(gr, 16)],
                            oscores.at[pl.ds(b * _M + gr, 16)])
            pltpu.sync_copy(cbuf.at[pl.ds(gr, 16)],
                            oclasses.at[pl.ds(b * _M + gr, 16)])


_sc_kernel = functools.partial(
    pl.kernel,
    out_type=[
        jax.ShapeDtypeStruct((_B, 16), jnp.int32),
        jax.ShapeDtypeStruct((_T * 4,), jnp.float32),
        jax.ShapeDtypeStruct((_T,), jnp.float32),
        jax.ShapeDtypeStruct((_T,), jnp.int32),
    ],
    mesh=plsc.VectorSubcoreMesh(core_axis_name="c", subcore_axis_name="s"),
    compiler_params=pltpu.CompilerParams(needs_layout_passes=False),
    scratch_types=[
        pltpu.VMEM((_S + 16,), jnp.int32),
        pltpu.VMEM((_LDUMP + 32,), jnp.int32),
        pltpu.VMEM((8, 128), jnp.int32),
        pltpu.VMEM((8, 128), jnp.int32),
        pltpu.VMEM((_R,), jnp.int32),
        pltpu.VMEM((_R,), jnp.int32),
        pltpu.VMEM((_R,), jnp.int32),
        pltpu.VMEM((_R,), jnp.int32),
        pltpu.VMEM((_R,), jnp.float32),
        pltpu.VMEM((8, 2048), jnp.float32),
        pltpu.VMEM((8, 2048), jnp.float32),
        pltpu.VMEM((4, 2048), jnp.float32),
        pltpu.VMEM((4, 2048), jnp.float32),
        pltpu.VMEM((_R * 4,), jnp.float32),
        pltpu.VMEM((16,), jnp.int32),
        pltpu.SemaphoreType.DMA,
        pltpu.SemaphoreType.DMA,
        pltpu.SemaphoreType.DMA,
    ],
)(_sc_body)


def kernel(pred_boxes, pred_scores, selected_indexes):
    sel1d = selected_indexes.astype(jnp.int32).T.reshape(3 * _S)
    boxes_t = jnp.transpose(pred_boxes, (0, 2, 1))
    scores_t = jnp.transpose(pred_scores, (0, 2, 1))
    counts, oboxes, oscores, oclasses = _sc_kernel(sel1d, boxes_t, scores_t)
    return (counts[:, :1],
            oboxes.reshape(_B, _M, 4),
            oscores.reshape(_B, _M),
            oclasses.reshape(_B, _M))


# speedup vs baseline: 3.8709x; 1.0268x over previous
import functools

import jax
import jax.numpy as jnp
from jax import lax
from jax.experimental import pallas as pl
from jax.experimental.pallas import tpu as pltpu
from jax.experimental.pallas import tpu_sc as plsc

_B = 16
_N = 20000
_C = 80
_M = 1000
_S = 8000
_T = _B * _M
_R = 1024
_NG = _R // 16
_NCH = 32
_LDUMP = 1024


def _gather16(x, idx):
    dnums = lax.GatherDimensionNumbers(
        offset_dims=(), collapsed_slice_dims=(0,), start_index_map=(0,))
    return lax.gather(x, idx[:, None], dnums, (1,),
                      mode=lax.GatherScatterMode.PROMISE_IN_BOUNDS)


def _cumsum16(x):
    idx = lax.iota(jnp.int32, 16)
    y = x
    for sh in (1, 2, 4, 8):
        src = jnp.where(idx >= sh, idx - sh, 0)
        g = _gather16(y, src)
        y = y + jnp.where(idx >= sh, g, 0)
    return y


def _lane15(x):
    return _gather16(x, jnp.full((16,), 15, jnp.int32))


def _sc_body(sel1d, boxes_t, scores_t,
             counts, oboxes, oscores, oclasses,
             sel_b, list_v, xidx_v, lidx_v, xg, lg,
             pbuf, cbuf, sobuf, sstA, sstB, bstA, bstB, ibuf, cnt_ref,
             gsem, semA, semB):
    b = lax.axis_index("s")
    h = lax.axis_index("c")
    zero16 = jnp.zeros((16,), jnp.int32)
    zf16 = jnp.zeros((16,), jnp.float32)

    @pl.loop(0, _R // 16)
    def _z1(g):
        sobuf[pl.ds(g * 16, 16)] = zf16
        cbuf[pl.ds(g * 16, 16)] = zero16

    @pl.loop(0, 4 * _R // 16)
    def _z2(g):
        ibuf[pl.ds(g * 16, 16)] = zf16

    pltpu.sync_copy(sel1d.at[pl.ds(0, _S)], sel_b.at[pl.ds(0, _S)])

    @pl.loop(0, _LDUMP // 16)
    def _z3(g):
        list_v[pl.ds(g * 16, 16)] = zero16
    cnt_ref[...] = zero16

    base_iota = lax.iota(jnp.int32, 16)

    def _filter(g, cv):
        bv = sel_b[pl.ds(g * 16, 16)]
        m = bv == b
        mi = jnp.where(m, jnp.int32(1), jnp.int32(0))
        cs = _cumsum16(mi)
        pos = cv + cs - 1
        pos = jnp.where(m, pos, _S + base_iota)
        pos = jnp.minimum(pos, _LDUMP + base_iota)
        plsc.store_scatter(list_v, [pos], base_iota + g * 16)
        return cv + _lane15(cs)

    cnt_vec = lax.fori_loop(0, _S // 16, _filter, jnp.zeros((16,), jnp.int32))
    cnt_ref[...] = cnt_vec
    cap_vec = jnp.minimum(cnt_vec, _M)
    cap_s = cap_vec[0]

    @pl.loop(0, _NG)
    def _bidx(g):
        rid = list_v[pl.ds(g * 16, 16)]
        xidx_v[g >> 3, pl.ds((g & 7) * 16, 16)] = rid + 2 * _S
        lidx_v[g >> 3, pl.ds((g & 7) * 16, 16)] = rid + _S
    idx_copies = []
    for r in range(8):
        idx_copies.append(pltpu.async_copy(
            sel1d.at[xidx_v.at[r]], xg.at[pl.ds(r * 128, 128)], gsem))
        idx_copies.append(pltpu.async_copy(
            sel1d.at[lidx_v.at[r]], lg.at[pl.ds(r * 128, 128)], gsem))
    for cp in idx_copies:
        cp.wait()

    @pl.loop(0, _NG)
    def _pack(g):
        sl = pl.ds(g * 16, 16)
        x = xg[sl]
        l = lg[sl]
        rvec = g * 16 + base_iota
        valid = rvec < cap_vec
        pbuf[sl] = jnp.where(valid, l * 32768 + x, (rvec * 128) & 16383)
        cbuf[sl] = jnp.where(valid, l, 0)

    def issue(k, sst, bst, sem):
        gr = (2 * k + h) * 16
        pv = pbuf[pl.ds(gr, 16)]
        lv8 = (pv >> 15) & 120
        xv128 = ((pv & 32767) >> 7) * 128
        for j in range(16):
            l8 = pl.multiple_of(lv8[j], 8)
            x128 = pl.multiple_of(xv128[j], 128)
            pltpu.async_copy(scores_t.at[b, pl.ds(l8, 8), pl.ds(x128, 128)],
                             sst.at[:, pl.ds(j * 128, 128)], sem)
            pltpu.async_copy(boxes_t.at[b, :, pl.ds(x128, 128)],
                             bst.at[:, pl.ds(j * 128, 128)], sem)

    def drain(sst, bst, sem):
        pltpu.make_async_copy(
            scores_t.at[0, pl.ds(0, 8), pl.ds(0, 2048)], sst, sem).wait()
        pltpu.make_async_copy(
            boxes_t.at[0, :, pl.ds(0, 2048)], bst, sem).wait()

    col_ids = [jnp.full((16,), c, jnp.int32) for c in range(4)]

    def extract(k, sst, bst):
        gr = (2 * k + h) * 16
        p = pbuf[pl.ds(gr, 16)]
        x = p & 32767
        la = (p >> 15) & 7
        rvec = gr + base_iota
        invalid = rvec >= cap_vec
        spos = base_iota * 128 + (x & 127)
        sval = plsc.load_gather(sst, [la, spos])
        plsc.store_scatter(sobuf, [rvec], jnp.where(invalid, zf16, sval))
        for c in range(4):
            bval = plsc.load_gather(bst, [col_ids[c], spos])
            bval = jnp.where(invalid, zf16, bval)
            plsc.store_scatter(ibuf, [rvec * 4 + c], bval)

    def active(k):
        return (2 * k + h) * 16 < cap_s

    bufs = [(sstA, bstA, semA), (sstB, bstB, semB)]

    @pl.when(active(0))
    def _():
        issue(0, *bufs[0])

    for cc in range(_NCH):
        if cc + 1 < _NCH:
            @pl.when(active(cc + 1))
            def _(cc=cc):
                issue(cc + 1, *bufs[(cc + 1) % 2])

        @pl.when(active(cc))
        def _(cc=cc):
            drain(*bufs[cc % 2])
            extract(cc, bufs[cc % 2][0], bufs[cc % 2][1])

    out_copies = []

    @pl.when(h == 0)
    def _():
        for k in range(32):
            gr = 32 * k
            n = 16 if gr + 16 <= _M else _M - gr
            out_copies.append(pltpu.async_copy(
                ibuf.at[pl.ds(4 * gr, 4 * n)],
                oboxes.at[pl.ds(b * 4 * _M + 4 * gr, 4 * n)], gsem))
            out_copies.append(pltpu.async_copy(
                sobuf.at[pl.ds(gr, n)],
                oscores.at[pl.ds(b * _M + gr, n)], gsem))
            out_copies.append(pltpu.async_copy(
                cbuf.at[pl.ds(gr, n)],
                oclasses.at[pl.ds(b * _M + gr, n)], gsem))
        out_copies.append(pltpu.async_copy(cnt_ref, counts.at[b], gsem))
        for cp in out_copies:
            cp.wait()

    out_copies2 = []

    @pl.when(h == 1)
    def _():
        for k in range(31):
            gr = 32 * k + 16
            out_copies2.append(pltpu.async_copy(
                ibuf.at[pl.ds(4 * gr, 64)],
                oboxes.at[pl.ds(b * 4 * _M + 4 * gr, 64)], gsem))
            out_copies2.append(pltpu.async_copy(
                sobuf.at[pl.ds(gr, 16)],
                oscores.at[pl.ds(b * _M + gr, 16)], gsem))
            out_copies2.append(pltpu.async_copy(
                cbuf.at[pl.ds(gr, 16)],
                oclasses.at[pl.ds(b * _M + gr, 16)], gsem))
        for cp in out_copies2:
            cp.wait()


_sc_kernel = functools.partial(
    pl.kernel,
    out_type=[
        jax.ShapeDtypeStruct((_B, 16), jnp.int32),
        jax.ShapeDtypeStruct((_T * 4,), jnp.float32),
        jax.ShapeDtypeStruct((_T,), jnp.float32),
        jax.ShapeDtypeStruct((_T,), jnp.int32),
    ],
    mesh=plsc.VectorSubcoreMesh(core_axis_name="c", subcore_axis_name="s"),
    compiler_params=pltpu.CompilerParams(needs_layout_passes=False),
    scratch_types=[
        pltpu.VMEM((_S + 16,), jnp.int32),
        pltpu.VMEM((_LDUMP + 32,), jnp.int32),
        pltpu.VMEM((8, 128), jnp.int32),
        pltpu.VMEM((8, 128), jnp.int32),
        pltpu.VMEM((_R,), jnp.int32),
        pltpu.VMEM((_R,), jnp.int32),
        pltpu.VMEM((_R,), jnp.int32),
        pltpu.VMEM((_R,), jnp.int32),
        pltpu.VMEM((_R,), jnp.float32),
        pltpu.VMEM((8, 2048), jnp.float32),
        pltpu.VMEM((8, 2048), jnp.float32),
        pltpu.VMEM((4, 2048), jnp.float32),
        pltpu.VMEM((4, 2048), jnp.float32),
        pltpu.VMEM((_R * 4,), jnp.float32),
        pltpu.VMEM((16,), jnp.int32),
        pltpu.SemaphoreType.DMA,
        pltpu.SemaphoreType.DMA,
        pltpu.SemaphoreType.DMA,
    ],
)(_sc_body)


def kernel(pred_boxes, pred_scores, selected_indexes):
    sel1d = selected_indexes.astype(jnp.int32).T.reshape(3 * _S)
    boxes_t = jnp.transpose(pred_boxes, (0, 2, 1))
    scores_t = jnp.transpose(pred_scores, (0, 2, 1))
    counts, oboxes, oscores, oclasses = _sc_kernel(sel1d, boxes_t, scores_t)
    return (counts[:, :1],
            oboxes.reshape(_B, _M, 4),
            oscores.reshape(_B, _M),
            oclasses.reshape(_B, _M))


# speedup vs baseline: 4.1584x; 1.0743x over previous
import functools

import jax
import jax.numpy as jnp
from jax import lax
from jax.experimental import pallas as pl
from jax.experimental.pallas import tpu as pltpu
from jax.experimental.pallas import tpu_sc as plsc

_B = 16
_N = 20000
_C = 80
_M = 1000
_S = 8000
_T = _B * _M
_R = 1024
_NG = _R // 16
_NCH = 32
_LDUMP = 1024


def _gather16(x, idx):
    dnums = lax.GatherDimensionNumbers(
        offset_dims=(), collapsed_slice_dims=(0,), start_index_map=(0,))
    return lax.gather(x, idx[:, None], dnums, (1,),
                      mode=lax.GatherScatterMode.PROMISE_IN_BOUNDS)


def _cumsum16(x):
    idx = lax.iota(jnp.int32, 16)
    y = x
    for sh in (1, 2, 4, 8):
        src = jnp.where(idx >= sh, idx - sh, 0)
        g = _gather16(y, src)
        y = y + jnp.where(idx >= sh, g, 0)
    return y


def _lane15(x):
    return _gather16(x, jnp.full((16,), 15, jnp.int32))


def _sc_body(sel1d, boxes_t, scores_t,
             counts, oboxes, oscores, oclasses,
             sel_b, list_v, xidx_v, lidx_v, xg, lg,
             pbuf, cbuf, sobuf, sstA, sstB, bstA, bstB, ibuf,
             drain96, drain48, cnt_ref, gsem, semA, semB):
    b = lax.axis_index("s")
    h = lax.axis_index("c")
    zero16 = jnp.zeros((16,), jnp.int32)
    zf16 = jnp.zeros((16,), jnp.float32)

    @pl.loop(0, _R // 16)
    def _z1(g):
        sobuf[pl.ds(g * 16, 16)] = zf16
        cbuf[pl.ds(g * 16, 16)] = zero16

    @pl.loop(0, 4 * _R // 16)
    def _z2(g):
        ibuf[pl.ds(g * 16, 16)] = zf16

    pltpu.sync_copy(sel1d.at[pl.ds(0, _S)], sel_b.at[pl.ds(0, _S)])

    @pl.loop(0, _LDUMP // 16)
    def _z3(g):
        list_v[pl.ds(g * 16, 16)] = zero16
    cnt_ref[...] = zero16

    base_iota = lax.iota(jnp.int32, 16)

    def _filter(g, cv):
        bv = sel_b[pl.ds(g * 16, 16)]
        m = bv == b
        mi = jnp.where(m, jnp.int32(1), jnp.int32(0))
        cs = _cumsum16(mi)
        pos = cv + cs - 1
        pos = jnp.where(m, pos, _S + base_iota)
        pos = jnp.minimum(pos, _LDUMP + base_iota)
        plsc.store_scatter(list_v, [pos], base_iota + g * 16)
        return cv + _lane15(cs)

    cnt_vec = lax.fori_loop(0, _S // 16, _filter, jnp.zeros((16,), jnp.int32))
    cnt_ref[...] = cnt_vec
    cap_vec = jnp.minimum(cnt_vec, _M)
    cap_s = cap_vec[0]

    @pl.loop(0, _NG)
    def _bidx(g):
        rid = list_v[pl.ds(g * 16, 16)]
        xidx_v[g >> 3, pl.ds((g & 7) * 16, 16)] = rid + 2 * _S
        lidx_v[g >> 3, pl.ds((g & 7) * 16, 16)] = rid + _S
    idx_copies = []
    for r in range(8):
        idx_copies.append(pltpu.async_copy(
            sel1d.at[xidx_v.at[r]], xg.at[pl.ds(r * 128, 128)], gsem))
        idx_copies.append(pltpu.async_copy(
            sel1d.at[lidx_v.at[r]], lg.at[pl.ds(r * 128, 128)], gsem))
    for cp in idx_copies:
        cp.wait()

    @pl.loop(0, _NG)
    def _pack(g):
        sl = pl.ds(g * 16, 16)
        x = xg[sl]
        l = lg[sl]
        rvec = g * 16 + base_iota
        valid = rvec < cap_vec
        pbuf[sl] = jnp.where(valid, l * 32768 + x, (rvec * 128) & 16383)
        cbuf[sl] = jnp.where(valid, l, 0)

    def issue(k, sst, bst, sem):
        gr = (2 * k + h) * 16
        pv = pbuf[pl.ds(gr, 16)]
        lv8 = (pv >> 15) & 120
        xv128 = ((pv & 32767) >> 7) * 128
        for j in range(16):
            l8 = pl.multiple_of(lv8[j], 8)
            x128 = pl.multiple_of(xv128[j], 128)
            pltpu.async_copy(scores_t.at[b, pl.ds(l8, 8), pl.ds(x128, 128)],
                             sst.at[:, pl.ds(j * 128, 128)], sem)
            pltpu.async_copy(boxes_t.at[b, :, pl.ds(x128, 128)],
                             bst.at[:, pl.ds(j * 128, 128)], sem)

    def drain(sst, bst, sem):
        pltpu.make_async_copy(
            scores_t.at[0, pl.ds(0, 8), pl.ds(0, 2048)], sst, sem).wait()
        pltpu.make_async_copy(
            boxes_t.at[0, :, pl.ds(0, 2048)], bst, sem).wait()

    col_ids = [jnp.full((16,), c, jnp.int32) for c in range(4)]

    def extract(k, sst, bst):
        gr = (2 * k + h) * 16
        p = pbuf[pl.ds(gr, 16)]
        x = p & 32767
        la = (p >> 15) & 7
        rvec = gr + base_iota
        invalid = rvec >= cap_vec
        spos = base_iota * 128 + (x & 127)
        sval = plsc.load_gather(sst, [la, spos])
        plsc.store_scatter(sobuf, [rvec], jnp.where(invalid, zf16, sval))
        for c in range(4):
            bval = plsc.load_gather(bst, [col_ids[c], spos])
            bval = jnp.where(invalid, zf16, bval)
            plsc.store_scatter(ibuf, [rvec * 4 + c], bval)

    def active(k):
        return (2 * k + h) * 16 < cap_s

    @pl.when(active(0))
    def _():
        issue(0, sstA, bstA, semA)

    @pl.loop(0, _NCH // 2)
    def _pipe(kk):
        c0 = 2 * kk
        @pl.when(active(c0 + 1))
        def _():
            issue(c0 + 1, sstB, bstB, semB)

        @pl.when(active(c0))
        def _():
            drain(sstA, bstA, semA)
            extract(c0, sstA, bstA)

        @pl.when((kk < _NCH // 2 - 1) & active(c0 + 2))
        def _():
            issue(c0 + 2, sstA, bstA, semA)

        @pl.when(active(c0 + 1))
        def _():
            drain(sstB, bstB, semB)
            extract(c0 + 1, sstB, bstB)

    @pl.loop(0, 31)
    def _out(k):
        gr = 32 * k + 16 * h
        pltpu.async_copy(ibuf.at[pl.ds(4 * gr, 64)],
                         oboxes.at[pl.ds(b * 4 * _M + 4 * gr, 64)], gsem)
        pltpu.async_copy(sobuf.at[pl.ds(gr, 16)],
                         oscores.at[pl.ds(b * _M + gr, 16)], gsem)
        pltpu.async_copy(cbuf.at[pl.ds(gr, 16)],
                         oclasses.at[pl.ds(b * _M + gr, 16)], gsem)

    @pl.when(h == 0)
    def _():
        pltpu.async_copy(ibuf.at[pl.ds(4 * 992, 32)],
                         oboxes.at[pl.ds(b * 4 * _M + 4 * 992, 32)], gsem)
        pltpu.async_copy(sobuf.at[pl.ds(992, 8)],
                         oscores.at[pl.ds(b * _M + 992, 8)], gsem)
        pltpu.async_copy(cbuf.at[pl.ds(992, 8)],
                         oclasses.at[pl.ds(b * _M + 992, 8)], gsem)
        pltpu.sync_copy(cnt_ref, counts.at[b])

    @pl.loop(0, 31)
    def _dr(k):
        pltpu.make_async_copy(
            oscores.at[pl.ds(0, 96)], drain96, gsem).wait()

    @pl.when(h == 0)
    def _():
        pltpu.make_async_copy(
            oscores.at[pl.ds(0, 48)], drain48, gsem).wait()


_sc_kernel = functools.partial(
    pl.kernel,
    out_type=[
        jax.ShapeDtypeStruct((_B, 16), jnp.int32),
        jax.ShapeDtypeStruct((_T * 4,), jnp.float32),
        jax.ShapeDtypeStruct((_T,), jnp.float32),
        jax.ShapeDtypeStruct((_T,), jnp.int32),
    ],
    mesh=plsc.VectorSubcoreMesh(core_axis_name="c", subcore_axis_name="s"),
    compiler_params=pltpu.CompilerParams(needs_layout_passes=False),
    scratch_types=[
        pltpu.VMEM((_S + 16,), jnp.int32),
        pltpu.VMEM((_LDUMP + 32,), jnp.int32),
        pltpu.VMEM((8, 128), jnp.int32),
        pltpu.VMEM((8, 128), jnp.int32),
        pltpu.VMEM((_R,), jnp.int32),
        pltpu.VMEM((_R,), jnp.int32),
        pltpu.VMEM((_R,), jnp.int32),
        pltpu.VMEM((_R,), jnp.int32),
        pltpu.VMEM((_R,), jnp.float32),
        pltpu.VMEM((8, 2048), jnp.float32),
        pltpu.VMEM((8, 2048), jnp.float32),
        pltpu.VMEM((4, 2048), jnp.float32),
        pltpu.VMEM((4, 2048), jnp.float32),
        pltpu.VMEM((_R * 4,), jnp.float32),
        pltpu.VMEM((96,), jnp.float32),
        pltpu.VMEM((48,), jnp.float32),
        pltpu.VMEM((16,), jnp.int32),
        pltpu.SemaphoreType.DMA,
        pltpu.SemaphoreType.DMA,
        pltpu.SemaphoreType.DMA,
    ],
)(_sc_body)


def kernel(pred_boxes, pred_scores, selected_indexes):
    sel1d = selected_indexes.astype(jnp.int32).T.reshape(3 * _S)
    boxes_t = jnp.transpose(pred_boxes, (0, 2, 1))
    scores_t = jnp.transpose(pred_scores, (0, 2, 1))
    counts, oboxes, oscores, oclasses = _sc_kernel(sel1d, boxes_t, scores_t)
    return (counts[:, :1],
            oboxes.reshape(_B, _M, 4),
            oscores.reshape(_B, _M),
            oclasses.reshape(_B, _M))


# speedup vs baseline: 4.2359x; 1.0186x over previous
import functools

import jax
import jax.numpy as jnp
from jax import lax
from jax.experimental import pallas as pl
from jax.experimental.pallas import tpu as pltpu
from jax.experimental.pallas import tpu_sc as plsc

_B = 16
_N = 20000
_C = 80
_M = 1000
_S = 8000
_T = _B * _M
_R = 1024
_NG = _R // 16
_NCH = 32
_LDUMP = 1024


def _gather16(x, idx):
    dnums = lax.GatherDimensionNumbers(
        offset_dims=(), collapsed_slice_dims=(0,), start_index_map=(0,))
    return lax.gather(x, idx[:, None], dnums, (1,),
                      mode=lax.GatherScatterMode.PROMISE_IN_BOUNDS)


def _cumsum16(x):
    idx = lax.iota(jnp.int32, 16)
    y = x
    for sh in (1, 2, 4, 8):
        src = jnp.where(idx >= sh, idx - sh, 0)
        g = _gather16(y, src)
        y = y + jnp.where(idx >= sh, g, 0)
    return y


def _lane15(x):
    return _gather16(x, jnp.full((16,), 15, jnp.int32))


def _sc_body(sel1d, boxes_t, scores_t,
             counts, oboxes, oscores, oclasses,
             sel_b, list_v, xidx_v, lidx_v, xg, lg,
             pbuf, cbuf, sobuf, sstA, sstB, bstA, bstB, ibuf,
             drain96, drain48, cnt_ref, gsem, semA, semB):
    b = lax.axis_index("s")
    h = lax.axis_index("c")
    zero16 = jnp.zeros((16,), jnp.int32)
    zf16 = jnp.zeros((16,), jnp.float32)

    @pl.loop(0, _R // 16)
    def _z1(g):
        sobuf[pl.ds(g * 16, 16)] = zf16
        cbuf[pl.ds(g * 16, 16)] = zero16

    @pl.loop(0, 4 * _R // 16)
    def _z2(g):
        ibuf[pl.ds(g * 16, 16)] = zf16

    pltpu.sync_copy(sel1d.at[pl.ds(0, _S)], sel_b.at[pl.ds(0, _S)])

    @pl.loop(0, _LDUMP // 16)
    def _z3(g):
        list_v[pl.ds(g * 16, 16)] = zero16
    cnt_ref[...] = zero16

    base_iota = lax.iota(jnp.int32, 16)

    def _filter(g, cv):
        bv = sel_b[pl.ds(g * 16, 16)]
        m = bv == b
        mi = jnp.where(m, jnp.int32(1), jnp.int32(0))
        cs = plsc.cumsum(mi)
        pos = cv + cs - 1
        pos = jnp.where(m, pos, _S + base_iota)
        pos = jnp.minimum(pos, _LDUMP + base_iota)
        plsc.store_scatter(list_v, [pos], base_iota + g * 16)
        return cv + _lane15(cs)

    cnt_vec = lax.fori_loop(0, _S // 16, _filter, jnp.zeros((16,), jnp.int32))
    cnt_ref[...] = cnt_vec
    cap_vec = jnp.minimum(cnt_vec, _M)
    cap_s = cap_vec[0]

    @pl.loop(0, _NG)
    def _bidx(g):
        rid = list_v[pl.ds(g * 16, 16)]
        xidx_v[g >> 3, pl.ds((g & 7) * 16, 16)] = rid + 2 * _S
        lidx_v[g >> 3, pl.ds((g & 7) * 16, 16)] = rid + _S
    idx_copies = []
    for r in range(8):
        idx_copies.append(pltpu.async_copy(
            sel1d.at[xidx_v.at[r]], xg.at[pl.ds(r * 128, 128)], gsem))
        idx_copies.append(pltpu.async_copy(
            sel1d.at[lidx_v.at[r]], lg.at[pl.ds(r * 128, 128)], gsem))
    for cp in idx_copies:
        cp.wait()

    @pl.loop(0, _NG)
    def _pack(g):
        sl = pl.ds(g * 16, 16)
        x = xg[sl]
        l = lg[sl]
        rvec = g * 16 + base_iota
        valid = rvec < cap_vec
        pbuf[sl] = jnp.where(valid, l * 32768 + x, (rvec * 128) & 16383)
        cbuf[sl] = jnp.where(valid, l, 0)

    def issue(k, sst, bst, sem):
        gr = (2 * k + h) * 16
        pv = pbuf[pl.ds(gr, 16)]
        lv8 = (pv >> 15) & 120
        xv128 = ((pv & 32767) >> 7) * 128
        for j in range(16):
            l8 = pl.multiple_of(lv8[j], 8)
            x128 = pl.multiple_of(xv128[j], 128)
            pltpu.async_copy(scores_t.at[b, pl.ds(l8, 8), pl.ds(x128, 128)],
                             sst.at[:, pl.ds(j * 128, 128)], sem)
            pltpu.async_copy(boxes_t.at[b, :, pl.ds(x128, 128)],
                             bst.at[:, pl.ds(j * 128, 128)], sem)

    def drain(sst, bst, sem):
        pltpu.make_async_copy(
            scores_t.at[0, pl.ds(0, 8), pl.ds(0, 2048)], sst, sem).wait()
        pltpu.make_async_copy(
            boxes_t.at[0, :, pl.ds(0, 2048)], bst, sem).wait()

    col_ids = [jnp.full((16,), c, jnp.int32) for c in range(4)]

    def extract(k, sst, bst):
        gr = (2 * k + h) * 16
        p = pbuf[pl.ds(gr, 16)]
        x = p & 32767
        la = (p >> 15) & 7
        rvec = gr + base_iota
        invalid = rvec >= cap_vec
        spos = base_iota * 128 + (x & 127)
        sval = plsc.load_gather(sst, [la, spos])
        plsc.store_scatter(sobuf, [rvec], jnp.where(invalid, zf16, sval))
        for c in range(4):
            bval = plsc.load_gather(bst, [col_ids[c], spos])
            bval = jnp.where(invalid, zf16, bval)
            plsc.store_scatter(ibuf, [rvec * 4 + c], bval)

    def active(k):
        return (2 * k + h) * 16 < cap_s

    @pl.when(active(0))
    def _():
        issue(0, sstA, bstA, semA)

    @pl.loop(0, _NCH // 2)
    def _pipe(kk):
        c0 = 2 * kk
        @pl.when(active(c0 + 1))
        def _():
            issue(c0 + 1, sstB, bstB, semB)

        @pl.when(active(c0))
        def _():
            drain(sstA, bstA, semA)
            extract(c0, sstA, bstA)

        @pl.when((kk < _NCH // 2 - 1) & active(c0 + 2))
        def _():
            issue(c0 + 2, sstA, bstA, semA)

        @pl.when(active(c0 + 1))
        def _():
            drain(sstB, bstB, semB)
            extract(c0 + 1, sstB, bstB)

    @pl.loop(0, 31)
    def _out(k):
        gr = 32 * k + 16 * h
        pltpu.async_copy(ibuf.at[pl.ds(4 * gr, 64)],
                         oboxes.at[pl.ds(b * 4 * _M + 4 * gr, 64)], gsem)
        pltpu.async_copy(sobuf.at[pl.ds(gr, 16)],
                         oscores.at[pl.ds(b * _M + gr, 16)], gsem)
        pltpu.async_copy(cbuf.at[pl.ds(gr, 16)],
                         oclasses.at[pl.ds(b * _M + gr, 16)], gsem)

    @pl.when(h == 0)
    def _():
        pltpu.async_copy(ibuf.at[pl.ds(4 * 992, 32)],
                         oboxes.at[pl.ds(b * 4 * _M + 4 * 992, 32)], gsem)
        pltpu.async_copy(sobuf.at[pl.ds(992, 8)],
                         oscores.at[pl.ds(b * _M + 992, 8)], gsem)
        pltpu.async_copy(cbuf.at[pl.ds(992, 8)],
                         oclasses.at[pl.ds(b * _M + 992, 8)], gsem)
        pltpu.sync_copy(cnt_ref, counts.at[b])

    @pl.loop(0, 31)
    def _dr(k):
        pltpu.make_async_copy(
            oscores.at[pl.ds(0, 96)], drain96, gsem).wait()

    @pl.when(h == 0)
    def _():
        pltpu.make_async_copy(
            oscores.at[pl.ds(0, 48)], drain48, gsem).wait()


_sc_kernel = functools.partial(
    pl.kernel,
    out_type=[
        jax.ShapeDtypeStruct((_B, 16), jnp.int32),
        jax.ShapeDtypeStruct((_T * 4,), jnp.float32),
        jax.ShapeDtypeStruct((_T,), jnp.float32),
        jax.ShapeDtypeStruct((_T,), jnp.int32),
    ],
    mesh=plsc.VectorSubcoreMesh(core_axis_name="c", subcore_axis_name="s"),
    compiler_params=pltpu.CompilerParams(needs_layout_passes=False),
    scratch_types=[
        pltpu.VMEM((_S + 16,), jnp.int32),
        pltpu.VMEM((_LDUMP + 32,), jnp.int32),
        pltpu.VMEM((8, 128), jnp.int32),
        pltpu.VMEM((8, 128), jnp.int32),
        pltpu.VMEM((_R,), jnp.int32),
        pltpu.VMEM((_R,), jnp.int32),
        pltpu.VMEM((_R,), jnp.int32),
        pltpu.VMEM((_R,), jnp.int32),
        pltpu.VMEM((_R,), jnp.float32),
        pltpu.VMEM((8, 2048), jnp.float32),
        pltpu.VMEM((8, 2048), jnp.float32),
        pltpu.VMEM((4, 2048), jnp.float32),
        pltpu.VMEM((4, 2048), jnp.float32),
        pltpu.VMEM((_R * 4,), jnp.float32),
        pltpu.VMEM((96,), jnp.float32),
        pltpu.VMEM((48,), jnp.float32),
        pltpu.VMEM((16,), jnp.int32),
        pltpu.SemaphoreType.DMA,
        pltpu.SemaphoreType.DMA,
        pltpu.SemaphoreType.DMA,
    ],
)(_sc_body)


def kernel(pred_boxes, pred_scores, selected_indexes):
    sel1d = selected_indexes.astype(jnp.int32).T.reshape(3 * _S)
    boxes_t = jnp.transpose(pred_boxes, (0, 2, 1))
    scores_t = jnp.transpose(pred_scores, (0, 2, 1))
    counts, oboxes, oscores, oclasses = _sc_kernel(sel1d, boxes_t, scores_t)
    return (counts[:, :1],
            oboxes.reshape(_B, _M, 4),
            oscores.reshape(_B, _M),
            oclasses.reshape(_B, _M))
